# baseline (speedup 1.0000x reference)
import jax
import jax.numpy as jnp
from jax import lax
from jax.experimental import pallas as pl
from jax.experimental.pallas import tpu as pltpu
from jax.experimental.pallas import tpu_sc as plsc

N = 10000
E = 160000
D = 16

NC = 2
NS = 16
NW = NC * NS
CHUNK = 2000
NCHUNK = E // CHUNK
CPW = -(-NCHUNK // NW)
ROWS_PER_TILE = N // NS

B2 = 6400
G2 = E // B2
B4 = 2000


def _gather_body(x_hbm, ei_hbm, xs_hbm, xd_hbm, idx_s, idx_d, rows_s,
                 rows_d, sem, sem2):
    cid = lax.axis_index("c")
    sid = lax.axis_index("s")
    wid = sid * NC + cid

    def body(j, carry):
        k = wid + j * NW

        @pl.when(k < NCHUNK)
        def _():
            base = k * CHUNK
            pltpu.sync_copy(ei_hbm.at[0, k], idx_s)
            pltpu.sync_copy(ei_hbm.at[1, k], idx_d)
            g_s = pltpu.async_copy(x_hbm.at[idx_s], rows_s, sem)
            g_d = pltpu.async_copy(x_hbm.at[idx_d], rows_d, sem)
            g_s.wait()
            g_d.wait()
            w_s = pltpu.make_async_copy(rows_s, xs_hbm.at[pl.ds(base, CHUNK)],
                                        sem2)
            w_d = pltpu.make_async_copy(rows_d, xd_hbm.at[pl.ds(base, CHUNK)],
                                        sem2)
            w_s.start()
            w_d.start()
            w_s.wait()
            w_d.wait()

        return carry

    lax.fori_loop(0, CPW, body, 0)


def _scatter_body(vf_hbm, vr_hbm, ei_hbm, z_hbm, out_hbm, idx_s, idx_d,
                  val_f, val_r, sem, sem2, acc_sh):
    cid = lax.axis_index("c")
    sid = lax.axis_index("s")
    wid = sid * NC + cid

    row0 = sid * ROWS_PER_TILE
    pltpu.sync_copy(z_hbm.at[pl.ds(row0, ROWS_PER_TILE)],
                    acc_sh.at[pl.ds(row0, ROWS_PER_TILE)])
    plsc.subcore_barrier()

    def body(j, carry):
        k = wid + j * NW

        @pl.when(k < NCHUNK)
        def _():
            base = k * CHUNK
            pltpu.sync_copy(ei_hbm.at[1, k], idx_s)
            pltpu.sync_copy(ei_hbm.at[0, k], idx_d)
            l_f = pltpu.make_async_copy(vf_hbm.at[pl.ds(base, CHUNK)], val_f,
                                        sem2)
            l_r = pltpu.make_async_copy(vr_hbm.at[pl.ds(base, CHUNK)], val_r,
                                        sem2)
            l_f.start()
            l_r.start()
            l_f.wait()
            l_r.wait()
            s_f = pltpu.async_copy(val_f, acc_sh.at[idx_s], sem, add=True)
            s_r = pltpu.async_copy(val_r, acc_sh.at[idx_d], sem, add=True)
            s_f.wait()
            s_r.wait()

        return carry

    lax.fori_loop(0, CPW, body, 0)
    plsc.subcore_barrier()
    pltpu.sync_copy(acc_sh.at[pl.ds(row0, ROWS_PER_TILE)],
                    out_hbm.at[cid, pl.ds(row0, ROWS_PER_TILE)])


def _softplus(x):
    return jnp.maximum(x, 0.0) + jnp.log1p(jnp.exp(-jnp.abs(x)))


def _edge_body(alpha_ref, tf_ref, tr_ref, wf_ref, wr_ref, xs_ref, xd_ref,
               vf_ref, vr_ref):
    alpha = alpha_ref[0, 0]
    r1 = lax.broadcasted_iota(jnp.int32, (D * D, D), 0)
    c1 = lax.broadcasted_iota(jnp.int32, (D * D, D), 1)
    gm2 = jnp.where(r1 % D == c1, 1.0, 0.0)
    r2 = lax.broadcasted_iota(jnp.int32, (D, D * D), 0)
    c2 = lax.broadcasted_iota(jnp.int32, (D, D * D), 1)
    sm2 = jnp.where(c2 // D == r2, 1.0, 0.0)

    nt = (((1,), (1,)), ((), ()))

    xs = xs_ref[...]
    xd = xd_ref[...]

    def one_dir(tt, xin, xt, rw):
        w = alpha * _softplus(rw)
        xe = lax.dot_general(gm2, xin, nt, preferred_element_type=jnp.float32)
        m = jnp.dot(sm2, tt * xe, preferred_element_type=jnp.float32)
        val_t = w * (m - jnp.transpose(xt))
        return jnp.transpose(val_t)

    vf_ref[...] = one_dir(tf_ref[...], xs, xd, wf_ref[0])
    vr_ref[...] = one_dir(tr_ref[...], xd, xs, wr_ref[0])


def _combine_body(x_ref, deg_ref, p0_ref, p1_ref, out_ref):
    degf = deg_ref[0].astype(jnp.float32)
    r = jnp.transpose(1.0 / jnp.maximum(degf, 1.0))
    out_ref[...] = x_ref[...] + r * (p0_ref[0] + p1_ref[0])


def kernel(x, edge_index, deg, T, raw_w, alpha):
    ei3 = edge_index.reshape(2, NCHUNK, CHUNK)
    t_cols = T.transpose(1, 2, 0).reshape(D * D, 2 * E)
    rw3 = raw_w.reshape(2 * G2, 1, B2)
    deg3 = deg.reshape(N // B4, 1, B4)
    zeros_nd = jnp.zeros((N, D), jnp.float32)
    alpha2 = jnp.reshape(alpha, (1, 1))

    mesh = plsc.VectorSubcoreMesh(core_axis_name="c", subcore_axis_name="s")

    gather = pl.kernel(
        _gather_body,
        out_type=(jax.ShapeDtypeStruct((E, D), jnp.float32),
                  jax.ShapeDtypeStruct((E, D), jnp.float32)),
        mesh=mesh,
        compiler_params=pltpu.CompilerParams(use_tc_tiling_on_sc=False),
        scratch_types=[
            pltpu.VMEM((CHUNK,), jnp.int32),
            pltpu.VMEM((CHUNK,), jnp.int32),
            pltpu.VMEM((CHUNK, D), jnp.float32),
            pltpu.VMEM((CHUNK, D), jnp.float32),
            pltpu.SemaphoreType.DMA,
            pltpu.SemaphoreType.DMA,
        ],
    )
    xs, xd = gather(x, ei3)

    vf, vr = pl.pallas_call(
        _edge_body,
        grid=(G2,),
        in_specs=[
            pl.BlockSpec(memory_space=pltpu.SMEM),
            pl.BlockSpec((D * D, B2), lambda i: (0, i)),
            pl.BlockSpec((D * D, B2), lambda i: (0, i + G2)),
            pl.BlockSpec((1, 1, B2), lambda i: (i, 0, 0)),
            pl.BlockSpec((1, 1, B2), lambda i: (i + G2, 0, 0)),
            pl.BlockSpec((B2, D), lambda i: (i, 0)),
            pl.BlockSpec((B2, D), lambda i: (i, 0)),
        ],
        out_specs=[
            pl.BlockSpec((B2, D), lambda i: (i, 0)),
            pl.BlockSpec((B2, D), lambda i: (i, 0)),
        ],
        out_shape=[
            jax.ShapeDtypeStruct((E, D), jnp.float32),
            jax.ShapeDtypeStruct((E, D), jnp.float32),
        ],
    )(alpha2, t_cols, t_cols, rw3, rw3, xs, xd)

    scatter = pl.kernel(
        _scatter_body,
        out_type=jax.ShapeDtypeStruct((NC, N, D), jnp.float32),
        mesh=mesh,
        compiler_params=pltpu.CompilerParams(use_tc_tiling_on_sc=False),
        scratch_types=[
            pltpu.VMEM((CHUNK,), jnp.int32),
            pltpu.VMEM((CHUNK,), jnp.int32),
            pltpu.VMEM((CHUNK, D), jnp.float32),
            pltpu.VMEM((CHUNK, D), jnp.float32),
            pltpu.SemaphoreType.DMA,
            pltpu.SemaphoreType.DMA,
            pltpu.VMEM_SHARED((N, D), jnp.float32),
        ],
    )
    p = scatter(vf, vr, ei3, zeros_nd)

    out = pl.pallas_call(
        _combine_body,
        grid=(N // B4,),
        in_specs=[
            pl.BlockSpec((B4, D), lambda i: (i, 0)),
            pl.BlockSpec((1, 1, B4), lambda i: (i, 0, 0)),
            pl.BlockSpec((1, B4, D), lambda i: (0, i, 0)),
            pl.BlockSpec((1, B4, D), lambda i: (1, i, 0)),
        ],
        out_specs=pl.BlockSpec((B4, D), lambda i: (i, 0)),
        out_shape=jax.ShapeDtypeStruct((N, D), jnp.float32),
    )(x, deg3, p, p)
    return out


# speedup vs baseline: 1.0017x; 1.0017x over previous
import jax
import jax.numpy as jnp
from jax import lax
from jax.experimental import pallas as pl
from jax.experimental.pallas import tpu as pltpu
from jax.experimental.pallas import tpu_sc as plsc

N = 10000
E = 160000
D = 16

NC = 2
NS = 16
NW = NC * NS
CHUNK = 3200
NCHUNK = E // CHUNK
CPW = -(-NCHUNK // NW)
ROWS_PER_TILE = N // NS

B2 = 6400
G2 = E // B2
B4 = 2000


def _gather_body(x_hbm, ei_hbm, xs_hbm, xd_hbm, idx_s, idx_d, rows_s,
                 rows_d, sem, sem2):
    cid = lax.axis_index("c")
    sid = lax.axis_index("s")
    wid = sid * NC + cid

    def body(j, carry):
        k = wid + j * NW

        @pl.when(k < NCHUNK)
        def _():
            base = k * CHUNK
            pltpu.sync_copy(ei_hbm.at[0, k], idx_s)
            pltpu.sync_copy(ei_hbm.at[1, k], idx_d)
            g_s = pltpu.async_copy(x_hbm.at[idx_s], rows_s, sem)
            g_d = pltpu.async_copy(x_hbm.at[idx_d], rows_d, sem)
            g_s.wait()
            g_d.wait()
            w_s = pltpu.make_async_copy(rows_s, xs_hbm.at[pl.ds(base, CHUNK)],
                                        sem2)
            w_d = pltpu.make_async_copy(rows_d, xd_hbm.at[pl.ds(base, CHUNK)],
                                        sem2)
            w_s.start()
            w_d.start()
            w_s.wait()
            w_d.wait()

        return carry

    lax.fori_loop(0, CPW, body, 0)


def _scatter_body(vf_hbm, vr_hbm, ei_hbm, z_hbm, out_hbm, idx_s, idx_d,
                  val_f, val_r, sem, sem2, acc_sh):
    cid = lax.axis_index("c")
    sid = lax.axis_index("s")
    wid = sid * NC + cid

    row0 = sid * ROWS_PER_TILE
    pltpu.sync_copy(z_hbm.at[pl.ds(row0, ROWS_PER_TILE)],
                    acc_sh.at[pl.ds(row0, ROWS_PER_TILE)])
    plsc.subcore_barrier()

    def body(j, carry):
        k = wid + j * NW

        @pl.when(k < NCHUNK)
        def _():
            base = k * CHUNK
            pltpu.sync_copy(ei_hbm.at[1, k], idx_s)
            pltpu.sync_copy(ei_hbm.at[0, k], idx_d)
            l_f = pltpu.make_async_copy(vf_hbm.at[pl.ds(base, CHUNK)], val_f,
                                        sem2)
            l_r = pltpu.make_async_copy(vr_hbm.at[pl.ds(base, CHUNK)], val_r,
                                        sem2)
            l_f.start()
            l_r.start()
            l_f.wait()
            l_r.wait()
            s_f = pltpu.async_copy(val_f, acc_sh.at[idx_s], sem, add=True)
            s_r = pltpu.async_copy(val_r, acc_sh.at[idx_d], sem, add=True)
            s_f.wait()
            s_r.wait()

        return carry

    lax.fori_loop(0, CPW, body, 0)
    plsc.subcore_barrier()
    pltpu.sync_copy(acc_sh.at[pl.ds(row0, ROWS_PER_TILE)],
                    out_hbm.at[cid, pl.ds(row0, ROWS_PER_TILE)])


def _softplus(x):
    return jnp.maximum(x, 0.0) + jnp.log1p(jnp.exp(-jnp.abs(x)))


def _edge_body(alpha_ref, tf_ref, tr_ref, wf_ref, wr_ref, xs_ref, xd_ref,
               vf_ref, vr_ref):
    alpha = alpha_ref[0, 0]
    r1 = lax.broadcasted_iota(jnp.int32, (D * D, D), 0)
    c1 = lax.broadcasted_iota(jnp.int32, (D * D, D), 1)
    gm2 = jnp.where(r1 % D == c1, 1.0, 0.0)
    r2 = lax.broadcasted_iota(jnp.int32, (D, D * D), 0)
    c2 = lax.broadcasted_iota(jnp.int32, (D, D * D), 1)
    sm2 = jnp.where(c2 // D == r2, 1.0, 0.0)

    nt = (((1,), (1,)), ((), ()))

    xs = xs_ref[...]
    xd = xd_ref[...]

    def one_dir(tt, xin, xt, rw):
        w = alpha * _softplus(rw)
        xe = lax.dot_general(gm2, xin, nt, preferred_element_type=jnp.float32)
        m = jnp.dot(sm2, tt * xe, preferred_element_type=jnp.float32)
        val_t = w * (m - jnp.transpose(xt))
        return jnp.transpose(val_t)

    vf_ref[...] = one_dir(tf_ref[...], xs, xd, wf_ref[0])
    vr_ref[...] = one_dir(tr_ref[...], xd, xs, wr_ref[0])


def _combine_body(x_ref, deg_ref, p0_ref, p1_ref, out_ref):
    degf = deg_ref[0].astype(jnp.float32)
    r = jnp.transpose(1.0 / jnp.maximum(degf, 1.0))
    out_ref[...] = x_ref[...] + r * (p0_ref[0] + p1_ref[0])


def kernel(x, edge_index, deg, T, raw_w, alpha):
    ei3 = edge_index.reshape(2, NCHUNK, CHUNK)
    t_cols = T.transpose(1, 2, 0).reshape(D * D, 2 * E)
    rw3 = raw_w.reshape(2 * G2, 1, B2)
    deg3 = deg.reshape(N // B4, 1, B4)
    zeros_nd = jnp.zeros((N, D), jnp.float32)
    alpha2 = jnp.reshape(alpha, (1, 1))

    mesh = plsc.VectorSubcoreMesh(core_axis_name="c", subcore_axis_name="s")

    gather = pl.kernel(
        _gather_body,
        out_type=(jax.ShapeDtypeStruct((E, D), jnp.float32),
                  jax.ShapeDtypeStruct((E, D), jnp.float32)),
        mesh=mesh,
        compiler_params=pltpu.CompilerParams(use_tc_tiling_on_sc=False),
        scratch_types=[
            pltpu.VMEM((CHUNK,), jnp.int32),
            pltpu.VMEM((CHUNK,), jnp.int32),
            pltpu.VMEM((CHUNK, D), jnp.float32),
            pltpu.VMEM((CHUNK, D), jnp.float32),
            pltpu.SemaphoreType.DMA,
            pltpu.SemaphoreType.DMA,
        ],
    )
    xs, xd = gather(x, ei3)

    vf, vr = pl.pallas_call(
        _edge_body,
        grid=(G2,),
        in_specs=[
            pl.BlockSpec(memory_space=pltpu.SMEM),
            pl.BlockSpec((D * D, B2), lambda i: (0, i)),
            pl.BlockSpec((D * D, B2), lambda i: (0, i + G2)),
            pl.BlockSpec((1, 1, B2), lambda i: (i, 0, 0)),
            pl.BlockSpec((1, 1, B2), lambda i: (i + G2, 0, 0)),
            pl.BlockSpec((B2, D), lambda i: (i, 0)),
            pl.BlockSpec((B2, D), lambda i: (i, 0)),
        ],
        out_specs=[
            pl.BlockSpec((B2, D), lambda i: (i, 0)),
            pl.BlockSpec((B2, D), lambda i: (i, 0)),
        ],
        out_shape=[
            jax.ShapeDtypeStruct((E, D), jnp.float32),
            jax.ShapeDtypeStruct((E, D), jnp.float32),
        ],
    )(alpha2, t_cols, t_cols, rw3, rw3, xs, xd)

    scatter = pl.kernel(
        _scatter_body,
        out_type=jax.ShapeDtypeStruct((NC, N, D), jnp.float32),
        mesh=mesh,
        compiler_params=pltpu.CompilerParams(use_tc_tiling_on_sc=False),
        scratch_types=[
            pltpu.VMEM((CHUNK,), jnp.int32),
            pltpu.VMEM((CHUNK,), jnp.int32),
            pltpu.VMEM((CHUNK, D), jnp.float32),
            pltpu.VMEM((CHUNK, D), jnp.float32),
            pltpu.SemaphoreType.DMA,
            pltpu.SemaphoreType.DMA,
            pltpu.VMEM_SHARED((N, D), jnp.float32),
        ],
    )
    p = scatter(vf, vr, ei3, zeros_nd)

    out = pl.pallas_call(
        _combine_body,
        grid=(N // B4,),
        in_specs=[
            pl.BlockSpec((B4, D), lambda i: (i, 0)),
            pl.BlockSpec((1, 1, B4), lambda i: (i, 0, 0)),
            pl.BlockSpec((1, B4, D), lambda i: (0, i, 0)),
            pl.BlockSpec((1, B4, D), lambda i: (1, i, 0)),
        ],
        out_specs=pl.BlockSpec((B4, D), lambda i: (i, 0)),
        out_shape=jax.ShapeDtypeStruct((N, D), jnp.float32),
    )(x, deg3, p, p)
    return out


# speedup vs baseline: 1.0230x; 1.0213x over previous
import jax
import jax.numpy as jnp
from jax import lax
from jax.experimental import pallas as pl
from jax.experimental.pallas import tpu as pltpu
from jax.experimental.pallas import tpu_sc as plsc

N = 10000
E = 160000
D = 16

NC = 2
NS = 16
NW = NC * NS
CHUNK = 5000
NCHUNK = E // CHUNK
CPW = NCHUNK // NW
ROWS_PER_TILE = N // NS

B2 = 6400
G2 = E // B2
B4 = 2000


def _gather_body(x_hbm, ei_hbm, xs_hbm, xd_hbm, idx_v, rows_v, sem):
    cid = lax.axis_index("c")
    sid = lax.axis_index("s")
    wid = sid * NC + cid

    def body(j, carry):
        k = wid * CPW + j
        base = k * CHUNK
        pltpu.sync_copy(ei_hbm.at[0, k], idx_v)
        pltpu.async_copy(x_hbm.at[idx_v], rows_v, sem).wait()
        pltpu.sync_copy(rows_v, xs_hbm.at[pl.ds(base, CHUNK)])
        pltpu.sync_copy(ei_hbm.at[1, k], idx_v)
        pltpu.async_copy(x_hbm.at[idx_v], rows_v, sem).wait()
        pltpu.sync_copy(rows_v, xd_hbm.at[pl.ds(base, CHUNK)])
        return carry

    lax.fori_loop(0, CPW, body, 0)


def _scatter_body(vf_hbm, vr_hbm, ei_hbm, z_hbm, out_hbm, idx_v, val_v, sem,
                  acc_sh):
    cid = lax.axis_index("c")
    sid = lax.axis_index("s")
    wid = sid * NC + cid

    row0 = sid * ROWS_PER_TILE
    pltpu.sync_copy(z_hbm.at[pl.ds(row0, ROWS_PER_TILE)],
                    acc_sh.at[pl.ds(row0, ROWS_PER_TILE)])
    plsc.subcore_barrier()

    def body(j, carry):
        k = wid * CPW + j
        base = k * CHUNK
        pltpu.sync_copy(ei_hbm.at[1, k], idx_v)
        pltpu.sync_copy(vf_hbm.at[pl.ds(base, CHUNK)], val_v)
        pltpu.sync_copy(val_v, acc_sh.at[idx_v], add=True)
        pltpu.sync_copy(ei_hbm.at[0, k], idx_v)
        pltpu.sync_copy(vr_hbm.at[pl.ds(base, CHUNK)], val_v)
        pltpu.sync_copy(val_v, acc_sh.at[idx_v], add=True)
        return carry

    lax.fori_loop(0, CPW, body, 0)
    plsc.subcore_barrier()
    pltpu.sync_copy(acc_sh.at[pl.ds(row0, ROWS_PER_TILE)],
                    out_hbm.at[cid, pl.ds(row0, ROWS_PER_TILE)])


def _softplus(x):
    return jnp.maximum(x, 0.0) + jnp.log1p(jnp.exp(-jnp.abs(x)))


def _edge_body(alpha_ref, tf_ref, tr_ref, wf_ref, wr_ref, xs_ref, xd_ref,
               vf_ref, vr_ref):
    alpha = alpha_ref[0, 0]
    r1 = lax.broadcasted_iota(jnp.int32, (D * D, D), 0)
    c1 = lax.broadcasted_iota(jnp.int32, (D * D, D), 1)
    gm2 = jnp.where(r1 % D == c1, 1.0, 0.0)
    r2 = lax.broadcasted_iota(jnp.int32, (D, D * D), 0)
    c2 = lax.broadcasted_iota(jnp.int32, (D, D * D), 1)
    sm2 = jnp.where(c2 // D == r2, 1.0, 0.0)

    nt = (((1,), (1,)), ((), ()))

    xs = xs_ref[...]
    xd = xd_ref[...]

    def one_dir(tt, xin, xt, rw):
        w = alpha * _softplus(rw)
        xe = lax.dot_general(gm2, xin, nt, preferred_element_type=jnp.float32)
        m = jnp.dot(sm2, tt * xe, preferred_element_type=jnp.float32)
        val_t = w * (m - jnp.transpose(xt))
        return jnp.transpose(val_t)

    vf_ref[...] = one_dir(tf_ref[...], xs, xd, wf_ref[0])
    vr_ref[...] = one_dir(tr_ref[...], xd, xs, wr_ref[0])


def _combine_body(x_ref, deg_ref, p0_ref, p1_ref, out_ref):
    degf = deg_ref[0].astype(jnp.float32)
    r = jnp.transpose(1.0 / jnp.maximum(degf, 1.0))
    out_ref[...] = x_ref[...] + r * (p0_ref[0] + p1_ref[0])


def kernel(x, edge_index, deg, T, raw_w, alpha):
    ei3 = edge_index.reshape(2, NCHUNK, CHUNK)
    t_cols = T.transpose(1, 2, 0).reshape(D * D, 2 * E)
    rw3 = raw_w.reshape(2 * G2, 1, B2)
    deg3 = deg.reshape(N // B4, 1, B4)
    zeros_nd = jnp.zeros((N, D), jnp.float32)
    alpha2 = jnp.reshape(alpha, (1, 1))

    mesh = plsc.VectorSubcoreMesh(core_axis_name="c", subcore_axis_name="s")

    gather = pl.kernel(
        _gather_body,
        out_type=(jax.ShapeDtypeStruct((E, D), jnp.float32),
                  jax.ShapeDtypeStruct((E, D), jnp.float32)),
        mesh=mesh,
        compiler_params=pltpu.CompilerParams(use_tc_tiling_on_sc=False),
        scratch_types=[
            pltpu.VMEM((CHUNK,), jnp.int32),
            pltpu.VMEM((CHUNK, D), jnp.float32),
            pltpu.SemaphoreType.DMA,
        ],
    )
    xs, xd = gather(x, ei3)

    vf, vr = pl.pallas_call(
        _edge_body,
        grid=(G2,),
        in_specs=[
            pl.BlockSpec(memory_space=pltpu.SMEM),
            pl.BlockSpec((D * D, B2), lambda i: (0, i)),
            pl.BlockSpec((D * D, B2), lambda i: (0, i + G2)),
            pl.BlockSpec((1, 1, B2), lambda i: (i, 0, 0)),
            pl.BlockSpec((1, 1, B2), lambda i: (i + G2, 0, 0)),
            pl.BlockSpec((B2, D), lambda i: (i, 0)),
            pl.BlockSpec((B2, D), lambda i: (i, 0)),
        ],
        out_specs=[
            pl.BlockSpec((B2, D), lambda i: (i, 0)),
            pl.BlockSpec((B2, D), lambda i: (i, 0)),
        ],
        out_shape=[
            jax.ShapeDtypeStruct((E, D), jnp.float32),
            jax.ShapeDtypeStruct((E, D), jnp.float32),
        ],
    )(alpha2, t_cols, t_cols, rw3, rw3, xs, xd)

    scatter = pl.kernel(
        _scatter_body,
        out_type=jax.ShapeDtypeStruct((NC, N, D), jnp.float32),
        mesh=mesh,
        compiler_params=pltpu.CompilerParams(use_tc_tiling_on_sc=False),
        scratch_types=[
            pltpu.VMEM((CHUNK,), jnp.int32),
            pltpu.VMEM((CHUNK, D), jnp.float32),
            pltpu.SemaphoreType.DMA,
            pltpu.VMEM_SHARED((N, D), jnp.float32),
        ],
    )
    p = scatter(vf, vr, ei3, zeros_nd)

    out = pl.pallas_call(
        _combine_body,
        grid=(N // B4,),
        in_specs=[
            pl.BlockSpec((B4, D), lambda i: (i, 0)),
            pl.BlockSpec((1, 1, B4), lambda i: (i, 0, 0)),
            pl.BlockSpec((1, B4, D), lambda i: (0, i, 0)),
            pl.BlockSpec((1, B4, D), lambda i: (1, i, 0)),
        ],
        out_specs=pl.BlockSpec((B4, D), lambda i: (i, 0)),
        out_shape=jax.ShapeDtypeStruct((N, D), jnp.float32),
    )(x, deg3, p, p)
    return out
